# baseline (speedup 1.0000x reference)
import functools

import jax
import jax.numpy as jnp
from jax import lax
from jax.experimental import pallas as pl
from jax.experimental.pallas import tpu as pltpu
from jax.experimental.pallas import tpu_sc as plsc

_R, _C = 64, 32768
_NC, _NS, _L = 2, 16, 16
_NW = _NC * _NS
_SC_R = 16
_TC_R0 = _SC_R
_HALF = _C // 2
_CHUNK = 8192
_NCHUNK = _HALF // _CHUNK
_SB = 512
_WPR = _C // 32
_BCH = _CHUNK // 32

_mesh = plsc.VectorSubcoreMesh(core_axis_name="c", subcore_axis_name="s")


@functools.partial(
    pl.kernel,
    out_type=jax.ShapeDtypeStruct((_SC_R * _C,), jnp.float32),
    mesh=_mesh,
    scratch_types=[
        pltpu.VMEM((_NCHUNK, 1, _CHUNK), jnp.float32),
        pltpu.VMEM((_NCHUNK, 1, _BCH), jnp.int32),
        pltpu.SemaphoreType.DMA((_NCHUNK,)),
        pltpu.SemaphoreType.DMA((_NCHUNK,)),
        pltpu.SemaphoreType.DMA((_NCHUNK,)),
    ],
)
def _domain_mask_sc(w_hbm, b_hbm, out_hbm, w_v, b_v, s_w, s_b, s_o):
    wid = lax.axis_index("s") * _NC + lax.axis_index("c")
    row = wid // 2
    col0 = (wid % 2) * _HALF
    base = pl.multiple_of(row * _C + col0, 8192)
    bbase = pl.multiple_of(row * _WPR + col0 // 32, 256)

    zero = jnp.zeros((_L,), jnp.float32)

    in_w, in_b, out_h = [], [], []
    for c in range(_NCHUNK):
        col = col0 + c * _CHUNK
        in_w.append(pltpu.async_copy(
            w_hbm.at[pl.ds(row, 1), pl.ds(col, _CHUNK)], w_v.at[c], s_w.at[c]))
        in_b.append(pltpu.async_copy(
            b_hbm.at[pl.ds(bbase + c * _BCH, _BCH)],
            b_v.at[c, 0], s_b.at[c]))

    for c in range(_NCHUNK):
        in_w[c].wait()
        in_b[c].wait()

        @plsc.parallel_loop(0, _CHUNK // _SB)
        def _body(p):
            words = b_v[c, 0, pl.ds(p * _L, _L)]
            for q in range(32):
                off = p * _SB + q * _L
                vec = w_v[c, 0, pl.ds(off, _L)]
                hit = (words << (31 - q)) < 0
                w_v[c, 0, pl.ds(off, _L)] = jnp.where(hit, vec, zero)

        out_h.append(pltpu.async_copy(
            w_v.at[c, 0], out_hbm.at[pl.ds(base + c * _CHUNK, _CHUNK)],
            s_o.at[c]))

    for h in out_h:
        h.wait()


def _tc_body(w_ref, m_ref, o_ref):
    o_ref[...] = jnp.where(m_ref[...], w_ref[...], jnp.float32(0.0))


_tc_rows = pl.pallas_call(
    _tc_body,
    grid=((_R - _TC_R0) // 8,),
    in_specs=[
        pl.BlockSpec((8, _C), lambda i: (i + _TC_R0 // 8, 0)),
        pl.BlockSpec((8, _C), lambda i: (i + _TC_R0 // 8, 0)),
    ],
    out_specs=pl.BlockSpec((8, _C), lambda i: (i + _TC_R0 // 8, 0)),
    out_shape=jax.ShapeDtypeStruct((_R, _C), jnp.float32),
)


def _pack_mask(mask_rows):
    mb = mask_rows.reshape(_SC_R, _C // _SB, 32, _L).astype(jnp.uint32)
    weights = jnp.left_shift(
        jnp.uint32(1), jnp.arange(32, dtype=jnp.uint32)
    )[None, None, :, None]
    packed = (mb * weights).sum(axis=2, dtype=jnp.uint32)
    return lax.bitcast_convert_type(packed, jnp.int32).reshape(_SC_R * _WPR)


def kernel(w, mask):
    sc_out = _domain_mask_sc(w, _pack_mask(mask[:_SC_R]))
    tc_out = _tc_rows(w, mask)
    return lax.dynamic_update_slice(tc_out, sc_out.reshape(_SC_R, _C), (0, 0))


# speedup vs baseline: 1.5588x; 1.5588x over previous
import functools

import jax
import jax.numpy as jnp
from jax import lax
from jax.experimental import pallas as pl
from jax.experimental.pallas import tpu as pltpu
from jax.experimental.pallas import tpu_sc as plsc

_R, _C = 64, 32768
_NC, _NS, _L = 2, 16, 16
_NW = _NC * _NS
_SC_R = 16
_PANEL = _C // _NW
_CHUNK = _PANEL // 2
_NCHUNK = 2

_mesh = plsc.VectorSubcoreMesh(core_axis_name="c", subcore_axis_name="s")


@functools.partial(
    pl.kernel,
    out_type=jax.ShapeDtypeStruct((_SC_R, _C), jnp.float32),
    mesh=_mesh,
    scratch_types=[
        pltpu.VMEM((_NCHUNK, _SC_R, _CHUNK), jnp.float32),
        pltpu.VMEM((_NCHUNK, _CHUNK), jnp.int32),
        pltpu.SemaphoreType.DMA((_NCHUNK,)),
        pltpu.SemaphoreType.DMA((_NCHUNK,)),
        pltpu.SemaphoreType.DMA((_NCHUNK,)),
    ],
)
def _domain_mask_sc(w_hbm, p_hbm, out_hbm, w_v, b_v, s_w, s_b, s_o):
    wid = lax.axis_index("s") * _NC + lax.axis_index("c")
    col0 = pl.multiple_of(wid * _PANEL, _PANEL)

    zero = jnp.zeros((_L,), jnp.float32)

    in_w, in_b, out_h = [], [], []
    for c in range(_NCHUNK):
        col = col0 + c * _CHUNK
        in_w.append(pltpu.async_copy(
            w_hbm.at[pl.ds(0, _SC_R), pl.ds(col, _CHUNK)],
            w_v.at[c], s_w.at[c]))
        in_b.append(pltpu.async_copy(
            p_hbm.at[pl.ds(col, _CHUNK)], b_v.at[c], s_b.at[c]))

    for c in range(_NCHUNK):
        in_w[c].wait()
        in_b[c].wait()

        @plsc.parallel_loop(0, _CHUNK // _L)
        def _body(j):
            words = b_v[c, pl.ds(j * _L, _L)]
            for r in range(_SC_R):
                vec = w_v[c, r, pl.ds(j * _L, _L)]
                hit = (words << (31 - r)) < 0
                w_v[c, r, pl.ds(j * _L, _L)] = jnp.where(hit, vec, zero)

        out_h.append(pltpu.async_copy(
            w_v.at[c],
            out_hbm.at[pl.ds(0, _SC_R), pl.ds(col0 + c * _CHUNK, _CHUNK)],
            s_o.at[c]))

    for h in out_h:
        h.wait()


def _tc_body(w_ref, m_ref, o_ref):
    o_ref[...] = jnp.where(m_ref[...] != 0, w_ref[...], jnp.float32(0.0))


_TC_GRID = (_R - _SC_R) // 8

_tc_rows = pl.pallas_call(
    _tc_body,
    grid=(_TC_GRID,),
    in_specs=[
        pl.BlockSpec((8, _C), lambda i: (i + _SC_R // 8, 0)),
        pl.BlockSpec((8, _C), lambda i: (i + _SC_R // 8, 0)),
    ],
    out_specs=pl.BlockSpec((8, _C), lambda i: (i + _SC_R // 8, 0)),
    out_shape=jax.ShapeDtypeStruct((_R, _C), jnp.float32),
)


def _pack_rows(m8):
    wt = jnp.left_shift(
        jnp.int32(1), jnp.arange(_SC_R, dtype=jnp.int32)
    )[:, None]
    return jnp.sum(m8[:_SC_R].astype(jnp.int32) * wt, axis=0)


def kernel(w, mask):
    m8 = mask.view(jnp.int8)
    sc_out = _domain_mask_sc(w, _pack_rows(m8))
    tc_out = _tc_rows(w, m8)
    return lax.dynamic_update_slice(tc_out, sc_out, (0, 0))
